# baseline (speedup 1.0000x reference)
import functools

import jax
import jax.numpy as jnp
from jax import lax
from jax.experimental import pallas as pl
from jax.experimental.pallas import tpu as pltpu
from jax.experimental.pallas import tpu_sc as plsc

_N = 2048
_K = 512
_D = 64
_S = 32


def _argmin_body(x_ref, cb_ref, idx_ref):
    x = x_ref[...]
    cb = cb_ref[...]
    cn = jnp.sum(cb * cb, axis=1, keepdims=True)
    cbaug = jnp.concatenate([cb, cn], axis=1)
    xaug = jnp.concatenate(
        [-2.0 * x, jnp.ones((x.shape[0], 1), jnp.float32)], axis=1)
    d = lax.dot_general(xaug, cbaug, (((1,), (1,)), ((), ())),
                        precision=lax.Precision.HIGHEST,
                        preferred_element_type=jnp.float32)
    dmin = jnp.min(d, axis=1, keepdims=True)
    iota = lax.broadcasted_iota(jnp.int32, d.shape, 1)
    idx_ref[...] = jnp.min(jnp.where(d == dmin, iota, _K), axis=1)


def _bsq_body(x_ref, zq_ref, wp_ref, bp_ref, wr_ref, br_ref,
              rec_ref, codes_ref):
    x = x_ref[...]
    z_q = zq_ref[...]
    pq = x + (z_q - x)
    r = x - pq
    s = lax.dot_general(r, wp_ref[...], (((1,), (0,)), ((), ())),
                        preferred_element_type=jnp.float32) + bp_ref[...]
    codes = (s > 0).astype(jnp.float32)
    codes_ref[...] = codes
    q = 2.0 * codes - 1.0
    bsq = lax.dot_general(q, wr_ref[...], (((1,), (0,)), ((), ())),
                          preferred_element_type=jnp.float32) + br_ref[...]
    aq = r + (bsq - r)
    rec = pq + aq
    rec_ref[...] = x + (rec - x)


def _make_sc_gather():
    info = plsc.get_sparse_core_info()
    nw = info.num_cores * info.num_subcores
    bpw = _N // nw
    mesh = plsc.VectorSubcoreMesh(core_axis_name="c", subcore_axis_name="s")

    @functools.partial(
        pl.kernel, mesh=mesh,
        compiler_params=pltpu.CompilerParams(use_tc_tiling_on_sc=False),
        out_type=jax.ShapeDtypeStruct((_N, _D), jnp.float32),
        scratch_types=[
            pltpu.VMEM((bpw,), jnp.int32),
            pltpu.VMEM((bpw, _D), jnp.float32),
            pltpu.SemaphoreType.DMA,
        ],
    )
    def sc_gather(idx_hbm, table_hbm, out_hbm, idx_v, rows_v, sem):
        wid = lax.axis_index("s") * info.num_cores + lax.axis_index("c")
        base = wid * bpw
        pltpu.sync_copy(idx_hbm.at[pl.ds(base, bpw)], idx_v)
        pltpu.async_copy(table_hbm.at[idx_v], rows_v, sem).wait()
        pltpu.sync_copy(rows_v, out_hbm.at[pl.ds(base, bpw)])

    return sc_gather


def kernel(x, codebook, Wp, bp, Wr, br):
    B, T, D = x.shape
    N = B * T
    x2 = x.reshape(N, D)
    bp2 = bp.reshape(1, _S)
    br2 = br.reshape(1, _D)

    idx = pl.pallas_call(
        _argmin_body,
        out_shape=jax.ShapeDtypeStruct((N,), jnp.int32),
    )(x2, codebook)

    z_q = _make_sc_gather()(idx, codebook)

    rec, codes = pl.pallas_call(
        _bsq_body,
        out_shape=[
            jax.ShapeDtypeStruct((N, D), jnp.float32),
            jax.ShapeDtypeStruct((N, _S), jnp.float32),
        ],
    )(x2, z_q, Wp, bp2, Wr, br2)

    return (rec.reshape(B, T, D), idx.reshape(B, T), codes.reshape(B, T, _S))


# speedup vs baseline: 2.1589x; 2.1589x over previous
import jax
import jax.numpy as jnp
from jax import lax
from jax.experimental import pallas as pl

_TILE = 2048
_K = 512
_D = 64
_S = 32


def _fused_body(x_ref, cb_ref, wp_ref, bp_ref, wr_ref, br_ref,
                rec_ref, idx_ref, codes_ref):
    x = x_ref[...]
    cb = cb_ref[...]

    cn = jnp.sum(cb * cb, axis=1, keepdims=True)
    cbaug = jnp.concatenate([cb, cn], axis=1)
    xaug = jnp.concatenate(
        [-2.0 * x, jnp.ones((x.shape[0], 1), jnp.float32)], axis=1)
    dt = lax.dot_general(cbaug, xaug, (((1,), (1,)), ((), ())),
                         precision=lax.Precision.HIGHEST,
                         preferred_element_type=jnp.float32)

    dmin = jnp.min(dt, axis=0, keepdims=True)
    iota_k = lax.broadcasted_iota(jnp.int32, dt.shape, 0)
    idx = jnp.min(jnp.where(dt == dmin, iota_k, _K), axis=0)
    idx_ref[...] = idx
    iota = lax.broadcasted_iota(jnp.int32, (x.shape[0], _K), 1)

    cb_hi = cb.astype(jnp.bfloat16).astype(jnp.float32)
    cb_mid = (cb - cb_hi).astype(jnp.bfloat16).astype(jnp.float32)
    cb_lo = cb - cb_hi - cb_mid
    onehot = (iota == idx[:, None]).astype(jnp.float32)
    dn = (((1,), (0,)), ((), ()))
    z_q = (lax.dot_general(onehot, cb_hi, dn,
                           preferred_element_type=jnp.float32)
           + lax.dot_general(onehot, cb_mid, dn,
                             preferred_element_type=jnp.float32)
           + lax.dot_general(onehot, cb_lo, dn,
                             preferred_element_type=jnp.float32))

    pq = x + (z_q - x)
    r = x - pq
    s = lax.dot_general(r, wp_ref[...], (((1,), (0,)), ((), ())),
                        preferred_element_type=jnp.float32) + bp_ref[...]
    codes = (s > 0).astype(jnp.float32)
    codes_ref[...] = codes
    q = 2.0 * codes - 1.0
    bsq = lax.dot_general(q, wr_ref[...], (((1,), (0,)), ((), ())),
                          preferred_element_type=jnp.float32) + br_ref[...]
    aq = r + (bsq - r)
    rec = pq + aq
    rec_ref[...] = x + (rec - x)


def kernel(x, codebook, Wp, bp, Wr, br):
    B, T, D = x.shape
    N = B * T
    x2 = x.reshape(N, D)
    bp2 = bp.reshape(1, _S)
    br2 = br.reshape(1, _D)

    grid = (N // _TILE,)
    rec, idx, codes = pl.pallas_call(
        _fused_body,
        grid=grid,
        in_specs=[
            pl.BlockSpec((_TILE, D), lambda i: (i, 0)),
            pl.BlockSpec((_K, D), lambda i: (0, 0)),
            pl.BlockSpec((D, _S), lambda i: (0, 0)),
            pl.BlockSpec((1, _S), lambda i: (0, 0)),
            pl.BlockSpec((_S, D), lambda i: (0, 0)),
            pl.BlockSpec((1, D), lambda i: (0, 0)),
        ],
        out_specs=[
            pl.BlockSpec((_TILE, D), lambda i: (i, 0)),
            pl.BlockSpec((_TILE,), lambda i: (i,)),
            pl.BlockSpec((_TILE, _S), lambda i: (i, 0)),
        ],
        out_shape=[
            jax.ShapeDtypeStruct((N, D), jnp.float32),
            jax.ShapeDtypeStruct((N,), jnp.int32),
            jax.ShapeDtypeStruct((N, _S), jnp.float32),
        ],
    )(x2, codebook, Wp, bp2, Wr, br2)

    return (rec.reshape(B, T, D), idx.reshape(B, T), codes.reshape(B, T, _S))


# speedup vs baseline: 2.3078x; 1.0690x over previous
import jax
import jax.numpy as jnp
from jax import lax
from jax.experimental import pallas as pl

_TILE = 2048
_K = 512
_D = 64
_S = 32


def _fused_body(x_ref, cb_ref, wp_ref, bp_ref, wr_ref, br_ref,
                rec_ref, idx_ref, codes_ref):
    x = x_ref[...]
    cb = cb_ref[...]

    cn = jnp.sum(cb * cb, axis=1, keepdims=True)
    cbaug = jnp.concatenate([cb, cn], axis=1)
    xaug = jnp.concatenate(
        [-2.0 * x, jnp.ones((x.shape[0], 1), jnp.float32)], axis=1)
    dt = lax.dot_general(cbaug, xaug, (((1,), (1,)), ((), ())),
                         precision=lax.Precision.HIGHEST,
                         preferred_element_type=jnp.float32)

    dmin = jnp.min(dt, axis=0, keepdims=True)
    iota_k = lax.broadcasted_iota(
        jnp.int32, dt.shape, 0).astype(jnp.float32)
    idx_f = jnp.min(jnp.where(dt == dmin, iota_k, float(_K)), axis=0)
    idx = idx_f.astype(jnp.int32)
    idx_ref[...] = idx
    iota = lax.broadcasted_iota(jnp.int32, (x.shape[0], _K), 1)

    cb_hi = cb.astype(jnp.bfloat16).astype(jnp.float32)
    cb_mid = (cb - cb_hi).astype(jnp.bfloat16).astype(jnp.float32)
    cb_lo = cb - cb_hi - cb_mid
    cb3 = jnp.concatenate([cb_hi, cb_mid, cb_lo], axis=1)
    onehot = (iota == idx[:, None]).astype(jnp.float32)
    z3 = lax.dot_general(onehot, cb3, (((1,), (0,)), ((), ())),
                         preferred_element_type=jnp.float32)
    z_q = (z3[:, :_D] + z3[:, _D:2 * _D]) + z3[:, 2 * _D:]

    pq = x + (z_q - x)
    r = x - pq
    s = lax.dot_general(r, wp_ref[...], (((1,), (0,)), ((), ())),
                        preferred_element_type=jnp.float32) + bp_ref[...]
    codes = (s > 0).astype(jnp.float32)
    codes_ref[...] = codes
    q = 2.0 * codes - 1.0
    bsq = lax.dot_general(q, wr_ref[...], (((1,), (0,)), ((), ())),
                          preferred_element_type=jnp.float32) + br_ref[...]
    aq = r + (bsq - r)
    rec = pq + aq
    rec_ref[...] = x + (rec - x)


def kernel(x, codebook, Wp, bp, Wr, br):
    B, T, D = x.shape
    N = B * T
    x2 = x.reshape(N, D)
    bp2 = bp.reshape(1, _S)
    br2 = br.reshape(1, _D)

    grid = (N // _TILE,)
    rec, idx, codes = pl.pallas_call(
        _fused_body,
        grid=grid,
        in_specs=[
            pl.BlockSpec((_TILE, D), lambda i: (i, 0)),
            pl.BlockSpec((_K, D), lambda i: (0, 0)),
            pl.BlockSpec((D, _S), lambda i: (0, 0)),
            pl.BlockSpec((1, _S), lambda i: (0, 0)),
            pl.BlockSpec((_S, D), lambda i: (0, 0)),
            pl.BlockSpec((1, D), lambda i: (0, 0)),
        ],
        out_specs=[
            pl.BlockSpec((_TILE, D), lambda i: (i, 0)),
            pl.BlockSpec((_TILE,), lambda i: (i,)),
            pl.BlockSpec((_TILE, _S), lambda i: (i, 0)),
        ],
        out_shape=[
            jax.ShapeDtypeStruct((N, D), jnp.float32),
            jax.ShapeDtypeStruct((N,), jnp.int32),
            jax.ShapeDtypeStruct((N, _S), jnp.float32),
        ],
    )(x2, codebook, Wp, bp2, Wr, br2)

    return (rec.reshape(B, T, D), idx.reshape(B, T), codes.reshape(B, T, _S))
